# baseline (speedup 1.0000x reference)
import functools

import jax
import jax.numpy as jnp
from jax import lax
from jax.experimental import pallas as pl
from jax.experimental.pallas import tpu as pltpu
from jax.experimental.pallas import tpu_sc as plsc

VOCAB = 100000
EMBED = 100
SEQ = 200
BATCH = 4096

NC = 2
NS = 16
NW = NC * NS
BPW = BATCH // NW
TOK = BPW * SEQ
GCHUNK = 20

BV = 2000


def _q_body(emb_ref, wt_ref, q_ref):
    wd = (wt_ref[1] - wt_ref[0]).astype(jnp.bfloat16)
    r = lax.dot_general(
        emb_ref[...], wd, (((1,), (1,)), ((), ())),
        preferred_element_type=jnp.float32)
    q_ref[0] = r[:, :128]
    q_ref[1] = r[:, 128:]


def _build_q(emb_table, wt):
    wt_pad = jnp.concatenate(
        [wt, jnp.zeros((2, 256 - SEQ, EMBED), jnp.float32)], axis=1)
    return pl.pallas_call(
        _q_body,
        grid=(VOCAB // BV,),
        in_specs=[
            pl.BlockSpec((BV, EMBED), lambda i: (i, 0)),
            pl.BlockSpec((2, 256, EMBED), lambda i: (0, 0, 0)),
        ],
        out_specs=pl.BlockSpec((2, BV, 128), lambda i: (0, i, 0)),
        out_shape=jax.ShapeDtypeStruct((2, VOCAB, 128), jnp.float32),
    )(emb_table, wt)


def _sc_body(idx_hbm, q_hbm, d_hbm, idx_v, g_v, d_v, sem):
    wid = lax.axis_index("s") * NC + lax.axis_index("c")
    pltpu.sync_copy(idx_hbm.at[wid], idx_v)

    def burst(i, carry):
        g0 = i * GCHUNK
        handles = []
        for j in range(GCHUNK):
            g = g0 + j
            handles.append(pltpu.async_copy(
                q_hbm.at[idx_v.at[g]],
                g_v.at[pl.ds(g * BPW, BPW)],
                sem))
        for h in handles:
            h.wait()
        return carry

    lax.fori_loop(0, SEQ // GCHUNK, burst, 0)

    def red(s, accs):
        base = s * BPW
        return tuple(a + g_v[pl.ds(base + k * 16, 16)]
                     for k, a in enumerate(accs))

    accs = lax.fori_loop(
        0, SEQ, red,
        tuple(jnp.zeros((16,), jnp.float32) for _ in range(BPW // 16)))
    for k, a in enumerate(accs):
        d_v[pl.ds(k * 16, 16)] = a
    pltpu.sync_copy(d_v, d_hbm.at[pl.ds(wid * BPW, BPW)])


def _gather_reduce(idx_blocks, q_flat):
    mesh = plsc.VectorSubcoreMesh(core_axis_name="c", subcore_axis_name="s")
    kern = functools.partial(
        pl.kernel,
        out_type=jax.ShapeDtypeStruct((BATCH,), jnp.float32),
        mesh=mesh,
        scratch_types=[
            pltpu.VMEM((SEQ, BPW), jnp.int32),
            pltpu.VMEM((TOK,), jnp.float32),
            pltpu.VMEM((BPW,), jnp.float32),
            pltpu.SemaphoreType.DMA,
        ],
    )(_sc_body)
    return kern(idx_blocks, q_flat)


def _fin_body(b_ref, d_ref, o0_ref, o1_ref):
    dd = d_ref[...] + (b_ref[1] - b_ref[0])
    t = jnp.log1p(jnp.exp(-jnp.abs(dd)))
    o0_ref[...] = -(jnp.maximum(dd, 0.0) + t)
    o1_ref[...] = -(jnp.maximum(-dd, 0.0) + t)


def _finalize(d, b):
    rows = BATCH // 128
    o0, o1 = pl.pallas_call(
        _fin_body,
        in_specs=[
            pl.BlockSpec(memory_space=pltpu.SMEM),
            pl.BlockSpec((rows, 128), lambda: (0, 0)),
        ],
        out_specs=[
            pl.BlockSpec((rows, 128), lambda: (0, 0)),
            pl.BlockSpec((rows, 128), lambda: (0, 0)),
        ],
        out_shape=[
            jax.ShapeDtypeStruct((rows, 128), jnp.float32),
            jax.ShapeDtypeStruct((rows, 128), jnp.float32),
        ],
    )(b, d.reshape(rows, 128))
    return jnp.stack([o0.reshape(-1), o1.reshape(-1)], axis=-1)


def kernel(input_data, emb_table, W, b):
    wt = W.T.reshape(2, SEQ, EMBED)
    s_ar = jnp.arange(SEQ, dtype=jnp.int32)[None, :]
    flat_idx = (input_data.astype(jnp.int32) * 128 + s_ar
                + (s_ar >= 128) * (VOCAB * 128 - 128))
    idx_blocks = flat_idx.reshape(NW, BPW, SEQ).transpose(0, 2, 1)

    q = _build_q(emb_table.astype(jnp.bfloat16), wt)
    d = _gather_reduce(idx_blocks, q.reshape(2 * VOCAB * 128))
    return _finalize(d, b)


# speedup vs baseline: 1.0545x; 1.0545x over previous
import functools

import jax
import jax.numpy as jnp
from jax import lax
from jax.experimental import pallas as pl
from jax.experimental.pallas import tpu as pltpu
from jax.experimental.pallas import tpu_sc as plsc

VOCAB = 100000
EMBED = 100
SEQ = 200
BATCH = 4096

NC = 2
NS = 16
NW = NC * NS
BPW = BATCH // NW
TOK = BPW * SEQ
GCHUNK = 20

BV = 2000


def _rne16(x):
    return (x + jnp.uint32(0x7FFF) + ((x >> 16) & jnp.uint32(1))) >> 16


def _q_body(emb_ref, wt_ref, q_ref):
    wd = wt_ref[1] - wt_ref[0]
    r = lax.dot_general(
        emb_ref[...], wd, (((1,), (1,)), ((), ())),
        preferred_element_type=jnp.float32)
    lo = _rne16(lax.bitcast_convert_type(r[:, :128], jnp.uint32))
    hi = _rne16(lax.bitcast_convert_type(r[:, 128:], jnp.uint32))
    q_ref[...] = (hi << 16) | lo


def _build_q(emb_table, wt):
    wt_pad = jnp.concatenate(
        [wt, jnp.zeros((2, 256 - SEQ, EMBED), jnp.float32)], axis=1)
    return pl.pallas_call(
        _q_body,
        grid=(VOCAB // BV,),
        in_specs=[
            pl.BlockSpec((BV, EMBED), lambda i: (i, 0)),
            pl.BlockSpec((2, 256, EMBED), lambda i: (0, 0, 0)),
        ],
        out_specs=pl.BlockSpec((BV, 128), lambda i: (i, 0)),
        out_shape=jax.ShapeDtypeStruct((VOCAB, 128), jnp.uint32),
    )(emb_table, wt_pad)


def _sc_body(idx_hbm, q_hbm, d_hbm, idx_v, g_v, d_v, sem):
    wid = lax.axis_index("s") * NC + lax.axis_index("c")
    pltpu.sync_copy(idx_hbm.at[wid], idx_v)

    def burst(i, carry):
        g0 = i * GCHUNK
        handles = []
        for j in range(GCHUNK):
            g = g0 + j
            handles.append(pltpu.async_copy(
                q_hbm.at[idx_v.at[g]],
                g_v.at[pl.ds(g * BPW, BPW)],
                sem))
        for h in handles:
            h.wait()
        return carry

    lax.fori_loop(0, SEQ // GCHUNK, burst, 0)

    def red_lo(s, accs):
        base = s * BPW
        return tuple(
            a + lax.bitcast_convert_type(g_v[pl.ds(base + k * 16, 16)] << 16,
                                         jnp.float32)
            for k, a in enumerate(accs))

    def red_hi(s, accs):
        base = s * BPW
        return tuple(
            a + lax.bitcast_convert_type(
                g_v[pl.ds(base + k * 16, 16)] & jnp.uint32(0xFFFF0000),
                jnp.float32)
            for k, a in enumerate(accs))

    zeros = tuple(jnp.zeros((16,), jnp.float32) for _ in range(BPW // 16))
    accs = lax.fori_loop(0, 128, red_lo, zeros)
    accs = lax.fori_loop(128, SEQ, red_hi, accs)
    for k, a in enumerate(accs):
        d_v[pl.ds(k * 16, 16)] = a
    pltpu.sync_copy(d_v, d_hbm.at[pl.ds(wid * BPW, BPW)])


def _gather_reduce(idx_blocks, q_flat):
    mesh = plsc.VectorSubcoreMesh(core_axis_name="c", subcore_axis_name="s")
    kern = functools.partial(
        pl.kernel,
        out_type=jax.ShapeDtypeStruct((BATCH,), jnp.float32),
        mesh=mesh,
        scratch_types=[
            pltpu.VMEM((SEQ, BPW), jnp.int32),
            pltpu.VMEM((TOK,), jnp.uint32),
            pltpu.VMEM((BPW,), jnp.float32),
            pltpu.SemaphoreType.DMA,
        ],
    )(_sc_body)
    return kern(idx_blocks, q_flat)


def _fin_body(b_ref, d_ref, o0_ref, o1_ref):
    dd = d_ref[...] + (b_ref[1] - b_ref[0])
    t = jnp.log1p(jnp.exp(-jnp.abs(dd)))
    o0_ref[...] = -(jnp.maximum(dd, 0.0) + t)
    o1_ref[...] = -(jnp.maximum(-dd, 0.0) + t)


def _finalize(d, b):
    rows = BATCH // 128
    o0, o1 = pl.pallas_call(
        _fin_body,
        in_specs=[
            pl.BlockSpec(memory_space=pltpu.SMEM),
            pl.BlockSpec((rows, 128), lambda: (0, 0)),
        ],
        out_specs=[
            pl.BlockSpec((rows, 128), lambda: (0, 0)),
            pl.BlockSpec((rows, 128), lambda: (0, 0)),
        ],
        out_shape=[
            jax.ShapeDtypeStruct((rows, 128), jnp.float32),
            jax.ShapeDtypeStruct((rows, 128), jnp.float32),
        ],
    )(b, d.reshape(rows, 128))
    return jnp.stack([o0.reshape(-1), o1.reshape(-1)], axis=-1)


def kernel(input_data, emb_table, W, b):
    wt = W.T.reshape(2, SEQ, EMBED)
    s_ar = jnp.arange(SEQ, dtype=jnp.int32)[None, :]
    flat_idx = input_data.astype(jnp.int32) * 128 + (s_ar & 127)
    idx_blocks = flat_idx.reshape(NW, BPW, SEQ).transpose(0, 2, 1)

    q = _build_q(emb_table, wt)
    d = _gather_reduce(idx_blocks, q.reshape(VOCAB * 128))
    return _finalize(d, b)
